# Initial kernel scaffold; baseline (speedup 1.0000x reference)
import jax
import jax.numpy as jnp
from jax.experimental import pallas as pl


def kernel(x, edge_index, Wl, bl, Wr, Wlin, blin):
    raise NotImplementedError("write your pallas kernel here")


# speedup vs baseline: 28.4914x; 28.4914x over previous
import functools

import jax
import jax.numpy as jnp
from jax import lax
from jax.experimental import pallas as pl
from jax.experimental.pallas import tpu as pltpu
from jax.experimental.pallas import tpu_sc as plsc

N = 100000
E = 3200000
D = 16
NCORE = 2
NSUB = 16
NW = NCORE * NSUB
CH = 128
K = 8
GROUP = CH * K
EPW = 100352
EP = EPW * NW
G = EPW // GROUP
NPAD = 100352
RPT = NPAD // NSUB
XR = N // 8
AR = NPAD // 8
BLK = 512
GRID = (XR + BLK - 1) // BLK


def _make_sc(with_deg):
    mesh = plsc.VectorSubcoreMesh(core_axis_name="c", subcore_axis_name="s")
    if with_deg:
        out_type = (
            jax.ShapeDtypeStruct((NCORE, NPAD, D), jnp.float32),
            jax.ShapeDtypeStruct((NCORE, NPAD), jnp.float32),
        )
    else:
        out_type = jax.ShapeDtypeStruct((NCORE, NPAD, D), jnp.float32)

    scratch = [
        pltpu.VMEM_SHARED((NPAD, D), jnp.float32),
        pltpu.VMEM((K, CH), jnp.int32),
        pltpu.VMEM((K, CH), jnp.int32),
        pltpu.VMEM((GROUP, D), jnp.float32),
        pltpu.SemaphoreType.DMA,
        pltpu.SemaphoreType.DMA,
    ]
    if with_deg:
        scratch += [
            pltpu.VMEM_SHARED((NPAD,), jnp.float32),
            pltpu.VMEM((CH,), jnp.float32),
        ]

    def body(*args):
        if with_deg:
            (x_hbm, src_hbm, dst_hbm, z2_hbm, z1_hbm, agg_out, deg_out,
             agg_sh, srci, dsti, rows, gsem, ssem, deg_sh, ones_v) = args
        else:
            (x_hbm, src_hbm, dst_hbm, z2_hbm, agg_out,
             agg_sh, srci, dsti, rows, gsem, ssem) = args
        c = lax.axis_index("c")
        s = lax.axis_index("s")
        t0 = s * RPT
        pltpu.sync_copy(z2_hbm.at[pl.ds(t0, RPT)], agg_sh.at[pl.ds(t0, RPT)])
        if with_deg:
            pltpu.sync_copy(z1_hbm.at[pl.ds(t0, RPT)], deg_sh.at[pl.ds(t0, RPT)])
            for i in range(CH // 16):
                ones_v[pl.ds(i * 16, 16)] = jnp.ones((16,), jnp.float32)
        plsc.subcore_barrier()

        wid = c * NSUB + s
        row0 = wid * (EPW // CH)

        def group(g, carry):
            grow = row0 + g * K
            pltpu.sync_copy(src_hbm.at[pl.ds(grow, K)], srci)
            pltpu.sync_copy(dst_hbm.at[pl.ds(grow, K)], dsti)
            gh = [pltpu.async_copy(x_hbm.at[srci.at[j]],
                                   rows.at[pl.ds(j * CH, CH)], gsem)
                  for j in range(K)]
            for h in gh:
                h.wait()
            sh = []
            for j in range(K):
                sh.append(pltpu.async_copy(rows.at[pl.ds(j * CH, CH)],
                                           agg_sh.at[dsti.at[j]], ssem,
                                           add=True))
                if with_deg:
                    sh.append(pltpu.async_copy(ones_v, deg_sh.at[dsti.at[j]],
                                               ssem, add=True))
            for h in sh:
                h.wait()
            return carry

        lax.fori_loop(0, G, group, 0)
        plsc.subcore_barrier()
        pltpu.sync_copy(agg_sh.at[pl.ds(t0, RPT)],
                        agg_out.at[c, pl.ds(t0, RPT)])
        if with_deg:
            pltpu.sync_copy(deg_sh.at[pl.ds(t0, RPT)],
                            deg_out.at[c, pl.ds(t0, RPT)])

    return pl.kernel(body, out_type=out_type, mesh=mesh,
                     scratch_types=scratch,
                     compiler_params=pltpu.CompilerParams(
                         use_tc_tiling_on_sc=False))


_sc_deg = _make_sc(True)
_sc_nodeg = _make_sc(False)


def _tc_body(x_ref, a0_ref, a1_ref, d0_ref, d1_ref, s_ref, w1_ref, w2_ref,
             b_ref, o_ref):
    agg = a0_ref[...] + a1_ref[...]
    deg = jnp.maximum(d0_ref[...] + d1_ref[...], 1.0)
    dot = functools.partial(jnp.dot, preferred_element_type=jnp.float32,
                            precision=lax.Precision.HIGHEST)
    dpk = dot(1.0 / deg, s_ref[...])
    o_ref[...] = (dot(agg * dpk, w1_ref[...]) + dot(x_ref[...], w2_ref[...])
                  + b_ref[...])


_tc = pl.pallas_call(
    _tc_body,
    grid=(GRID,),
    in_specs=[
        pl.BlockSpec((BLK, 128), lambda i: (i, 0)),
        pl.BlockSpec((BLK, 128), lambda i: (i, 0)),
        pl.BlockSpec((BLK, 128), lambda i: (i, 0)),
        pl.BlockSpec((BLK, 8), lambda i: (i, 0)),
        pl.BlockSpec((BLK, 8), lambda i: (i, 0)),
        pl.BlockSpec((8, 128), lambda i: (0, 0)),
        pl.BlockSpec((128, 128), lambda i: (0, 0)),
        pl.BlockSpec((128, 128), lambda i: (0, 0)),
        pl.BlockSpec((1, 128), lambda i: (0, 0)),
    ],
    out_specs=pl.BlockSpec((BLK, 128), lambda i: (i, 0)),
    out_shape=jax.ShapeDtypeStruct((XR, 128), jnp.float32),
)


def kernel(x, edge_index, Wl, bl, Wr, Wlin, blin):
    L = Wl.shape[0]
    pad = EP - E
    srcp = jnp.concatenate(
        [edge_index[0], jnp.zeros((pad,), jnp.int32)]).reshape(EP // CH, CH)
    dstp = jnp.concatenate(
        [edge_index[1], jnp.full((pad,), N, jnp.int32)]).reshape(EP // CH, CH)
    z2 = jnp.zeros((NPAD, D), jnp.float32)
    z1 = jnp.zeros((NPAD,), jnp.float32)

    eye8 = jnp.eye(8, dtype=jnp.float32)
    S = jnp.kron(eye8, jnp.ones((1, 16), jnp.float32))
    w1b = [jnp.kron(eye8, Wl[l].T) for l in range(L)]
    w2b = [jnp.kron(eye8, (Wr[l] + Wlin[l]).T) for l in range(L)]
    bpk = [jnp.tile(bl[l] + blin[l], 8).reshape(1, 128) for l in range(L)]

    xcur = x
    d0 = d1 = None
    for l in range(L):
        if l == 0:
            aggp, degp = _sc_deg(xcur, srcp, dstp, z2, z1)
            d0 = degp[0].reshape(AR, 8)
            d1 = degp[1].reshape(AR, 8)
        else:
            aggp = _sc_nodeg(xcur, srcp, dstp, z2)
        a0 = aggp[0].reshape(AR, 128)
        a1 = aggp[1].reshape(AR, 128)
        xp = _tc(xcur.reshape(XR, 128), a0, a1, d0, d1, S, w1b[l], w2b[l],
                 bpk[l])
        xcur = xp.reshape(N, D)
    return xcur


# speedup vs baseline: 42.5812x; 1.4945x over previous
import functools

import jax
import jax.numpy as jnp
from jax import lax
from jax.experimental import pallas as pl
from jax.experimental.pallas import tpu as pltpu
from jax.experimental.pallas import tpu_sc as plsc

N = 100000
E = 3200000
D = 16
NCORE = 2
NSUB = 16
NW = NCORE * NSUB
CH = 128
K = 8
ROWS = E // CH
RPW = ROWS // NW
G = RPW // K
TK = RPW - G * K
XTRA = ROWS - RPW * NW
NPAD = 100352
RPT = NPAD // NSUB
AR = NPAD // 8
XR = N // 8
BLK = 512
GRID = (AR + BLK - 1) // BLK


def _make_sc(with_deg):
    mesh = plsc.VectorSubcoreMesh(core_axis_name="c", subcore_axis_name="s")
    agg_t = jax.ShapeDtypeStruct((NPAD, D), jnp.float32)
    deg_t = jax.ShapeDtypeStruct((NCORE, NPAD), jnp.float32)
    out_type = (agg_t, agg_t, deg_t) if with_deg else (agg_t, agg_t)

    scratch = [
        pltpu.VMEM_SHARED((NPAD, D), jnp.float32),
        pltpu.VMEM((K, CH), jnp.int32),
        pltpu.VMEM((K, CH), jnp.int32),
        pltpu.VMEM((K * CH, D), jnp.float32),
        pltpu.SemaphoreType.DMA,
        pltpu.SemaphoreType.DMA,
    ]
    if with_deg:
        scratch += [
            pltpu.VMEM_SHARED((NPAD,), jnp.float32),
            pltpu.VMEM((CH,), jnp.float32),
        ]

    def body(*args):
        if with_deg:
            (xp_hbm, src_hbm, dst_hbm, z2_hbm, z1_hbm, a0_out, a1_out,
             d_out,
             agg_sh, srci, dsti, rows, gsem, ssem, deg_sh, ones_v) = args
        else:
            (xp_hbm, src_hbm, dst_hbm, z2_hbm, a0_out, a1_out,
             agg_sh, srci, dsti, rows, gsem, ssem) = args
        c = lax.axis_index("c")
        s = lax.axis_index("s")
        x_rows = xp_hbm
        t0 = s * RPT
        pltpu.sync_copy(z2_hbm.at[pl.ds(t0, RPT)], agg_sh.at[pl.ds(t0, RPT)])
        if with_deg:
            pltpu.sync_copy(z1_hbm.at[pl.ds(t0, RPT)], deg_sh.at[pl.ds(t0, RPT)])
            for i in range(CH // 16):
                ones_v[pl.ds(i * 16, 16)] = jnp.ones((16,), jnp.float32)
        plsc.subcore_barrier()

        wid = c * NSUB + s
        row0 = wid * RPW

        def do_chunks(nchunk):
            gh = [pltpu.async_copy(x_rows.at[srci.at[j]],
                                   rows.at[pl.ds(j * CH, CH)], gsem)
                  for j in range(nchunk)]
            for h in gh:
                h.wait()
            sh = []
            for j in range(nchunk):
                sh.append(pltpu.async_copy(
                    rows.at[pl.ds(j * CH, CH)],
                    agg_sh.at[dsti.at[j]], ssem, add=True))
                if with_deg:
                    sh.append(pltpu.async_copy(
                        ones_v, deg_sh.at[dsti.at[j]], ssem, add=True))
            for h in sh:
                h.wait()

        def group(g, carry):
            row = row0 + g * K
            pltpu.sync_copy(src_hbm.at[pl.ds(row, K)], srci)
            pltpu.sync_copy(dst_hbm.at[pl.ds(row, K)], dsti)
            do_chunks(K)
            return carry

        lax.fori_loop(0, G, group, 0)

        trow = row0 + G * K
        pltpu.sync_copy(src_hbm.at[pl.ds(trow, TK)], srci.at[pl.ds(0, TK)])
        pltpu.sync_copy(dst_hbm.at[pl.ds(trow, TK)], dsti.at[pl.ds(0, TK)])
        do_chunks(TK)

        @pl.when(wid < XTRA)
        def _():
            xrow = NW * RPW + wid
            pltpu.sync_copy(src_hbm.at[pl.ds(xrow, 1)], srci.at[pl.ds(0, 1)])
            pltpu.sync_copy(dst_hbm.at[pl.ds(xrow, 1)], dsti.at[pl.ds(0, 1)])
            do_chunks(1)

        plsc.subcore_barrier()
        aggv = agg_sh.at[pl.ds(t0, RPT)]
        if with_deg:
            pltpu.sync_copy(deg_sh.at[pl.ds(t0, RPT)],
                            d_out.at[c, pl.ds(t0, RPT)])

        @pl.when(c == 0)
        def _():
            pltpu.sync_copy(aggv, a0_out.at[pl.ds(t0, RPT)])

        @pl.when(c == 1)
        def _():
            pltpu.sync_copy(aggv, a1_out.at[pl.ds(t0, RPT)])

    return pl.kernel(body, out_type=out_type, mesh=mesh,
                     scratch_types=scratch,
                     compiler_params=pltpu.CompilerParams(
                         use_tc_tiling_on_sc=False))


_sc_deg = _make_sc(True)
_sc_nodeg = _make_sc(False)


def _tc_body(x_ref, a0_ref, a1_ref, d0_ref, d1_ref, s_ref, w1_ref, w2_ref,
             b_ref, o_ref):
    agg = a0_ref[...] + a1_ref[...]
    deg = jnp.maximum(d0_ref[...] + d1_ref[...], 1.0)
    dot = functools.partial(jnp.dot, preferred_element_type=jnp.float32,
                            precision=lax.Precision.HIGHEST)
    dpk = dot(1.0 / deg, s_ref[...])
    o_ref[...] = (dot(agg * dpk, w1_ref[...]) + dot(x_ref[...], w2_ref[...])
                  + b_ref[...])


_tc = pl.pallas_call(
    _tc_body,
    grid=(GRID,),
    in_specs=[
        pl.BlockSpec((BLK, 128), lambda i: (i, 0)),
        pl.BlockSpec((BLK, 128), lambda i: (i, 0)),
        pl.BlockSpec((BLK, 128), lambda i: (i, 0)),
        pl.BlockSpec((BLK, 8), lambda i: (i, 0)),
        pl.BlockSpec((BLK, 8), lambda i: (i, 0)),
        pl.BlockSpec((8, 128), lambda i: (0, 0)),
        pl.BlockSpec((128, 128), lambda i: (0, 0)),
        pl.BlockSpec((128, 128), lambda i: (0, 0)),
        pl.BlockSpec((1, 128), lambda i: (0, 0)),
    ],
    out_specs=pl.BlockSpec((BLK, 128), lambda i: (i, 0)),
    out_shape=jax.ShapeDtypeStruct((AR, 128), jnp.float32),
)


def kernel(x, edge_index, Wl, bl, Wr, Wlin, blin):
    L = Wl.shape[0]
    z2 = jnp.zeros((NPAD, D), jnp.float32)
    z1 = jnp.zeros((NPAD,), jnp.float32)

    eye8 = jnp.eye(8, dtype=jnp.float32)
    S = jnp.kron(eye8, jnp.ones((1, 16), jnp.float32))
    w1b = [jnp.kron(eye8, Wl[l].T) for l in range(L)]
    w2b = [jnp.kron(eye8, (Wr[l] + Wlin[l]).T) for l in range(L)]
    bpk = [jnp.tile(bl[l] + blin[l], 8).reshape(1, 128) for l in range(L)]

    xp = jnp.pad(x.reshape(XR, 128), ((0, AR - XR), (0, 0)))
    srcr = edge_index[0].reshape(ROWS, CH)
    dstr = edge_index[1].reshape(ROWS, CH)
    d0 = d1 = None
    for l in range(L):
        xrows = xp.reshape(NPAD, D)
        if l == 0:
            a0, a1, degp = _sc_deg(xrows, srcr, dstr, z2, z1)
            d0 = degp[0].reshape(AR, 8)
            d1 = degp[1].reshape(AR, 8)
        else:
            a0, a1 = _sc_nodeg(xrows, srcr, dstr, z2)
        xp = _tc(xp, a0.reshape(AR, 128), a1.reshape(AR, 128), d0, d1, S,
                 w1b[l], w2b[l], bpk[l])
    return xp[:XR].reshape(N, D)
